# baseline (speedup 1.0000x reference)
import functools

import jax
import jax.numpy as jnp
from jax import lax
from jax.experimental import pallas as pl
from jax.experimental.pallas import tpu as pltpu
from jax.experimental.pallas import tpu_sc as plsc

NC = 2
NS = 16
NW = NC * NS
L = 16

_SC_PARAMS = pltpu.CompilerParams(needs_layout_passes=False,
                                  use_tc_tiling_on_sc=True)


@functools.partial(jax.jit, static_argnames=("V", "D"))
def _sc_transpose(eu_t, ev_t, tail_u, tail_v, *, V, D):
    WB = 256
    NB = V // WB
    NBPW = (NB + NW - 1) // NW
    mesh = plsc.VectorSubcoreMesh(core_axis_name="c", subcore_axis_name="s")

    @functools.partial(
        pl.kernel,
        out_type=(jax.ShapeDtypeStruct((V // 2, 128), jnp.float32),
                  jax.ShapeDtypeStruct((V // 2, 128), jnp.float32)),
        mesh=mesh,
        compiler_params=_SC_PARAMS,
        scratch_types=[
            pltpu.VMEM((D, WB), jnp.float32),
            pltpu.VMEM((D, WB), jnp.float32),
            pltpu.VMEM((WB // 2, 128), jnp.float32),
            pltpu.VMEM((WB // 2, 128), jnp.float32),
            pltpu.SemaphoreType.DMA,
            pltpu.SemaphoreType.DMA,
        ],
    )
    def k(eu_hbm, ev_hbm, tlu_hbm, tlv_hbm, tu_hbm, tv_hbm,
          in0, in1, o0, o1, sin, sout):
        wid = lax.axis_index("s") * NC + lax.axis_index("c")
        iota = lax.iota(jnp.int32, L)

        def transpose_block(src, dst, nw):
            def grp(g, carry):
                wl = g * L + iota
                prow = wl >> 1
                pcol = (wl & 1) << 6
                for dd in range(D):
                    rd = (dd + iota) & (D - 1)
                    val = plsc.load_gather(src, [rd, wl])
                    plsc.store_scatter(dst, [prow, pcol + rd], val)
                return carry

            lax.fori_loop(0, nw // L, grp, 0)

        for t_hbm, o_hbm in ((eu_hbm, tu_hbm), (ev_hbm, tv_hbm)):
            start = wid * NBPW
            end = jnp.minimum(start + NBPW, NB)
            npair = (end - start + 1) // 2

            def pair(i, carry):
                b0 = start + 2 * i
                b1 = jnp.minimum(b0 + 1, end - 1)
                off0 = pl.multiple_of(b0 * WB, WB)
                off1 = pl.multiple_of(b1 * WB, WB)
                d0 = pltpu.async_copy(
                    t_hbm.at[:, pl.ds(off0, WB)], in0, sin)
                d1 = pltpu.async_copy(
                    t_hbm.at[:, pl.ds(off1, WB)], in1, sin)
                d0.wait()
                transpose_block(in0, o0, WB)
                w0 = pltpu.async_copy(
                    o0, o_hbm.at[pl.ds(pl.multiple_of(b0 * (WB // 2),
                                                      WB // 2), WB // 2)],
                    sout)
                d1.wait()
                transpose_block(in1, o1, WB)
                w1 = pltpu.async_copy(
                    o1, o_hbm.at[pl.ds(pl.multiple_of(b1 * (WB // 2),
                                                      WB // 2), WB // 2)],
                    sout)
                w0.wait()
                w1.wait()
                return carry

            lax.fori_loop(0, npair, pair, 0)

        @pl.when(wid == NW - 1)
        def _():
            pltpu.sync_copy(tlv_hbm, o0.at[pl.ds(0, 32)])
            pltpu.sync_copy(o0.at[pl.ds(0, 32)],
                            tv_hbm.at[pl.ds(V // 2 - 32, 32)])

        @pl.when(wid == NW - 2)
        def _():
            pltpu.sync_copy(tlu_hbm, o0.at[pl.ds(0, 32)])
            pltpu.sync_copy(o0.at[pl.ds(0, 32)],
                            tu_hbm.at[pl.ds(V // 2 - 32, 32)])

    return k(eu_t, ev_t, tail_u, tail_v)


@functools.partial(jax.jit, static_argnames=("B", "K", "D"))
def _sc_scores(pu2, par_u, pv2, par_v, ng2, par_ng, tu, tv, *, B, K, D):
    RPW = B // NW
    NHC = RPW * K // 128
    mesh = plsc.VectorSubcoreMesh(core_axis_name="c", subcore_axis_name="s")

    @functools.partial(
        pl.kernel,
        out_type=(jax.ShapeDtypeStruct((NW * 8, 128), jnp.float32),
                  jax.ShapeDtypeStruct((NW * 80, 128), jnp.float32)),
        mesh=mesh,
        compiler_params=_SC_PARAMS,
        scratch_types=[
            pltpu.VMEM((8, 128), jnp.int32),
            pltpu.VMEM((8, 128), jnp.int32),
            pltpu.VMEM((80, 128), jnp.int32),
            pltpu.VMEM((8, 128), jnp.int32),
            pltpu.VMEM((8, 128), jnp.int32),
            pltpu.VMEM((40, 128), jnp.int32),
            pltpu.VMEM((RPW, D), jnp.float32),
            pltpu.VMEM((128, 128), jnp.float32),
            pltpu.VMEM((128, 128), jnp.float32),
            pltpu.VMEM((8, 128), jnp.float32),
            pltpu.VMEM((40, 128), jnp.float32),
            pltpu.SemaphoreType.DMA,
            pltpu.SemaphoreType.DMA,
        ],
    )
    def k(pu_hbm, pup_hbm, pv_hbm, pvp_hbm, ng_hbm, ngp_hbm, tu_hbm, tv_hbm,
          out_pos, out_neg,
          pu_idx, pv_idx, ng_idx, par_u, par_v, par_ng,
          u_rows, buf_a, buf_b, s_pos, s_neg, sem_a, sem_b):
        wid = lax.axis_index("s") * NC + lax.axis_index("c")
        iota = lax.iota(jnp.int32, L)
        lrow_u = (wid & 1) * 4

        pltpu.sync_copy(pu_hbm.at[pl.ds(pl.multiple_of((wid >> 1) * 8, 8),
                                        8)], pu_idx)
        pltpu.sync_copy(pv_hbm.at[pl.ds(pl.multiple_of((wid >> 1) * 8, 8),
                                        8)], pv_idx)
        pltpu.sync_copy(ng_hbm.at[pl.ds(pl.multiple_of(wid * 80, 8), 80)],
                        ng_idx)
        pltpu.sync_copy(pup_hbm, par_u)
        pltpu.sync_copy(pvp_hbm, par_v)
        pltpu.sync_copy(ngp_hbm.at[pl.ds(pl.multiple_of((wid >> 3) * 40, 8),
                                         40)], par_ng)

        def par16(buf, row, col):
            pw = plsc.load_gather(buf, [jnp.full((L,), row, jnp.int32),
                                        jnp.full((L,), col, jnp.int32)])
            return ((pw >> iota) & 1) << 6

        def buf_sem(buf):
            return sem_a if buf is buf_a else sem_b

        def round_issue(t_hbm, idx_ref, row, buf):
            return pltpu.async_copy(t_hbm.at[idx_ref.at[row]], buf,
                                    buf_sem(buf))

        def round_wait(t_hbm, idx_ref, row, buf):
            pltpu.make_async_copy(t_hbm.at[idx_ref.at[row]], buf,
                                  buf_sem(buf)).wait()

        def u_round_compact(r, buf):
            def grp(g, carry):
                s_loc = g * L + iota
                gu = wid * 32 + r * 8 + g
                p16 = par16(par_u, gu >> 7, gu & 127)

                def dstep(d4, carry):
                    for u in range(4):
                        rd = (d4 * 4 + u + iota) & (D - 1)
                        val = plsc.load_gather(buf, [s_loc, p16 + rd])
                        plsc.store_scatter(u_rows, [r * 128 + s_loc, rd],
                                           val)
                    return carry

                lax.fori_loop(0, D // 4, dstep, 0)
                return carry

            lax.fori_loop(0, 8, grp, 0)

        round_issue(tu_hbm, pu_idx, lrow_u + 0, buf_a)
        round_issue(tu_hbm, pu_idx, lrow_u + 1, buf_b)
        round_wait(tu_hbm, pu_idx, lrow_u + 0, buf_a)
        u_round_compact(0, buf_a)
        round_wait(tu_hbm, pu_idx, lrow_u + 1, buf_b)
        u_round_compact(1, buf_b)
        round_issue(tu_hbm, pu_idx, lrow_u + 2, buf_a)
        round_issue(tu_hbm, pu_idx, lrow_u + 3, buf_b)
        round_wait(tu_hbm, pu_idx, lrow_u + 2, buf_a)
        u_round_compact(2, buf_a)
        round_wait(tu_hbm, pu_idx, lrow_u + 3, buf_b)
        u_round_compact(3, buf_b)

        def dot_pass(buf, urow_of, p16_of, sref, srow):
            def blk(j, carry):
                s_loc = j * L + iota
                urows = urow_of(j, s_loc)
                p16 = p16_of(j)

                def dstep(d4, accs):
                    out = []
                    for u in range(4):
                        rd = (d4 * 4 + u + iota) & (D - 1)
                        uc = plsc.load_gather(u_rows, [urows, rd])
                        oc = plsc.load_gather(buf, [s_loc, p16 + rd])
                        out.append(accs[u] + uc * oc)
                    return tuple(out)

                accs = lax.fori_loop(
                    0, D // 4, dstep,
                    tuple(jnp.zeros((L,), jnp.float32) for _ in range(4)))
                acc = (accs[0] + accs[1]) + (accs[2] + accs[3])
                plsc.store_scatter(sref,
                                   [jnp.full((L,), 0, jnp.int32) + srow,
                                    j * L + iota], acc)
                return carry

            lax.fori_loop(0, 8, blk, 0)

        def pos_quarter(buf, q):
            def p16_of(j):
                gv = wid * 32 + q * 8 + j
                return par16(par_v, gv >> 7, gv & 127)

            dot_pass(buf, lambda j, s_loc: q * 128 + s_loc, p16_of,
                     s_pos, q)

        round_issue(tv_hbm, pv_idx, lrow_u + 0, buf_a)
        round_issue(tv_hbm, pv_idx, lrow_u + 1, buf_b)
        round_wait(tv_hbm, pv_idx, lrow_u + 0, buf_a)
        pos_quarter(buf_a, 0)
        round_issue(tv_hbm, pv_idx, lrow_u + 2, buf_a)
        round_wait(tv_hbm, pv_idx, lrow_u + 1, buf_b)
        pos_quarter(buf_b, 1)
        round_issue(tv_hbm, pv_idx, lrow_u + 3, buf_b)
        round_wait(tv_hbm, pv_idx, lrow_u + 2, buf_a)
        pos_quarter(buf_a, 2)
        round_issue(tv_hbm, ng_idx, 0, buf_a)
        round_wait(tv_hbm, pv_idx, lrow_u + 3, buf_b)
        pos_quarter(buf_b, 3)
        pad = jnp.full((L,), 40.0, jnp.float32)
        for r in range(4, 8):
            for c in range(8):
                plsc.store_scatter(s_pos,
                                   [jnp.full((L,), r, jnp.int32),
                                    c * L + iota], pad)
        pltpu.sync_copy(s_pos,
                        out_pos.at[pl.ds(pl.multiple_of(wid * 8, 8), 8)])

        def neg_compute(buf, hc):
            def p16_of(j):
                g_w = hc * 8 + j
                return par16(par_ng, (wid & 7) * 5 + (g_w >> 7), g_w & 127)

            dot_pass(buf, lambda j, s_loc: (hc * 128 + s_loc) // K,
                     p16_of, s_neg, hc % 40)

        def pair(i, carry):
            hc0 = 2 * i
            hc1 = hc0 + 1
            round_issue(tv_hbm, ng_idx, hc1, buf_b)
            round_wait(tv_hbm, ng_idx, hc0, buf_a)
            neg_compute(buf_a, hc0)
            hc2 = jnp.minimum(hc0 + 2, NHC - 2)
            round_issue(tv_hbm, ng_idx, hc2, buf_a)
            round_wait(tv_hbm, ng_idx, hc1, buf_b)
            neg_compute(buf_b, hc1)

            @pl.when(hc1 == NHC // 2 - 1)
            def _():
                pltpu.sync_copy(
                    s_neg,
                    out_neg.at[pl.ds(pl.multiple_of(wid * 80, 8), 40)])

            return carry

        lax.fori_loop(0, NHC // 2, pair, 0)
        round_wait(tv_hbm, ng_idx, NHC - 2, buf_a)
        pltpu.sync_copy(
            s_neg,
            out_neg.at[pl.ds(pl.multiple_of(wid * 80 + 40, 8), 40)])

    return k(pu2, par_u, pv2, par_v, ng2, par_ng, tu, tv)


@functools.partial(jax.jit, static_argnames=("B",))
def _tc_loss(pos_score, neg_score, *, B):
    def body(p_ref, n_ref, o_ref):
        def log_sig(x):
            return jnp.minimum(x, 0.0) - jnp.log1p(jnp.exp(-jnp.abs(x)))

        tot = jnp.sum(log_sig(p_ref[...])) + jnp.sum(log_sig(-n_ref[...]))
        o_ref[0, 0] = -tot / B

    out = pl.pallas_call(
        body,
        out_shape=jax.ShapeDtypeStruct((1, 1), jnp.float32),
        out_specs=pl.BlockSpec(memory_space=pltpu.SMEM),
    )(pos_score, neg_score)
    return out[0, 0]


def _pack16(bits):
    w = bits.reshape(-1, 16) << jnp.arange(16, dtype=jnp.int32)
    return jnp.sum(w, axis=1).astype(jnp.int32)


def kernel(pos_u, pos_v, neg_v, embed_u, embed_v):
    V, D = embed_u.shape
    B, K = neg_v.shape
    ntail = (V % 256) // 2
    tu, tv = _sc_transpose(
        embed_u.T, embed_v.T,
        embed_u[V - 2 * ntail:, :].reshape(ntail, 2 * D),
        embed_v[V - 2 * ntail:, :].reshape(ntail, 2 * D), V=V, D=D)
    pu = pos_u.astype(jnp.int32)
    pv = pos_v.astype(jnp.int32)
    nf = neg_v.astype(jnp.int32).reshape(-1)
    pos_s, neg_s = _sc_scores(
        (pu >> 1).reshape(B // 128, 128), _pack16(pu & 1).reshape(-1, 128),
        (pv >> 1).reshape(B // 128, 128), _pack16(pv & 1).reshape(-1, 128),
        (nf >> 1).reshape(B * K // 128, 128),
        _pack16(nf & 1).reshape(-1, 128),
        tu, tv, B=B, K=K, D=D)
    return _tc_loss(pos_s, neg_s, B=B)


# speedup vs baseline: 2.1495x; 2.1495x over previous
import functools

import jax
import jax.numpy as jnp
from jax import lax
from jax.experimental import pallas as pl
from jax.experimental.pallas import tpu as pltpu
from jax.experimental.pallas import tpu_sc as plsc

NC = 2
NS = 16
NW = NC * NS
L = 16

_SC_PARAMS = pltpu.CompilerParams(needs_layout_passes=False,
                                  use_tc_tiling_on_sc=True)


@functools.partial(jax.jit, static_argnames=("V", "D"))
def _sc_transpose(eu_t, ev_t, tail_u, tail_v, *, V, D):
    WB = 256
    NB = V // WB
    NBPW = (NB + NW - 1) // NW
    mesh = plsc.VectorSubcoreMesh(core_axis_name="c", subcore_axis_name="s")

    @functools.partial(
        pl.kernel,
        out_type=(jax.ShapeDtypeStruct((V // 2, 128), jnp.float32),
                  jax.ShapeDtypeStruct((V // 2, 128), jnp.float32)),
        mesh=mesh,
        compiler_params=_SC_PARAMS,
        scratch_types=[
            pltpu.VMEM((D, WB), jnp.float32),
            pltpu.VMEM((D, WB), jnp.float32),
            pltpu.VMEM((WB // 2, 128), jnp.float32),
            pltpu.VMEM((WB // 2, 128), jnp.float32),
            pltpu.SemaphoreType.DMA,
            pltpu.SemaphoreType.DMA,
        ],
    )
    def k(eu_hbm, ev_hbm, tlu_hbm, tlv_hbm, tu_hbm, tv_hbm,
          in0, in1, o0, o1, sin, sout):
        wid = lax.axis_index("s") * NC + lax.axis_index("c")
        iota = lax.iota(jnp.int32, L)

        def transpose_block(src, dst, nw):
            def grp(g, carry):
                wl = g * L + iota
                prow = wl >> 1
                pcol = (wl & 1) << 6

                def dstep(dd):
                    rd = (dd + iota) & (D - 1)
                    val = plsc.load_gather(src, [rd, wl])
                    plsc.store_scatter(dst, [prow, pcol + rd], val)

                plsc.parallel_loop(0, D, unroll=8)(dstep)
                return carry

            lax.fori_loop(0, nw // L, grp, 0)

        for t_hbm, o_hbm in ((eu_hbm, tu_hbm), (ev_hbm, tv_hbm)):
            start = wid * NBPW
            end = jnp.minimum(start + NBPW, NB)
            npair = (end - start + 1) // 2

            def pair(i, carry):
                b0 = start + 2 * i
                b1 = jnp.minimum(b0 + 1, end - 1)
                off0 = pl.multiple_of(b0 * WB, WB)
                off1 = pl.multiple_of(b1 * WB, WB)
                d0 = pltpu.async_copy(
                    t_hbm.at[:, pl.ds(off0, WB)], in0, sin)
                d1 = pltpu.async_copy(
                    t_hbm.at[:, pl.ds(off1, WB)], in1, sin)
                d0.wait()
                transpose_block(in0, o0, WB)
                w0 = pltpu.async_copy(
                    o0, o_hbm.at[pl.ds(pl.multiple_of(b0 * (WB // 2),
                                                      WB // 2), WB // 2)],
                    sout)
                d1.wait()
                transpose_block(in1, o1, WB)
                w1 = pltpu.async_copy(
                    o1, o_hbm.at[pl.ds(pl.multiple_of(b1 * (WB // 2),
                                                      WB // 2), WB // 2)],
                    sout)
                w0.wait()
                w1.wait()
                return carry

            lax.fori_loop(0, npair, pair, 0)

        @pl.when(wid == NW - 1)
        def _():
            pltpu.sync_copy(tlv_hbm, o0.at[pl.ds(0, 32)])
            pltpu.sync_copy(o0.at[pl.ds(0, 32)],
                            tv_hbm.at[pl.ds(V // 2 - 32, 32)])

        @pl.when(wid == NW - 2)
        def _():
            pltpu.sync_copy(tlu_hbm, o0.at[pl.ds(0, 32)])
            pltpu.sync_copy(o0.at[pl.ds(0, 32)],
                            tu_hbm.at[pl.ds(V // 2 - 32, 32)])

    return k(eu_t, ev_t, tail_u, tail_v)


@functools.partial(jax.jit, static_argnames=("B", "K", "D"))
def _sc_scores(pu2, par_u, pv2, par_v, ng2, par_ng, tu, tv, *, B, K, D):
    RPW = B // NW
    NHC = RPW * K // 128
    mesh = plsc.VectorSubcoreMesh(core_axis_name="c", subcore_axis_name="s")

    @functools.partial(
        pl.kernel,
        out_type=(jax.ShapeDtypeStruct((NW * 8, 128), jnp.float32),
                  jax.ShapeDtypeStruct((NW * 80, 128), jnp.float32)),
        mesh=mesh,
        compiler_params=_SC_PARAMS,
        scratch_types=[
            pltpu.VMEM((8, 128), jnp.int32),
            pltpu.VMEM((8, 128), jnp.int32),
            pltpu.VMEM((80, 128), jnp.int32),
            pltpu.VMEM((8, 128), jnp.int32),
            pltpu.VMEM((8, 128), jnp.int32),
            pltpu.VMEM((40, 128), jnp.int32),
            pltpu.VMEM((RPW, D), jnp.float32),
            pltpu.VMEM((128, 128), jnp.float32),
            pltpu.VMEM((128, 128), jnp.float32),
            pltpu.VMEM((8, 128), jnp.float32),
            pltpu.VMEM((40, 128), jnp.float32),
            pltpu.SemaphoreType.DMA,
            pltpu.SemaphoreType.DMA,
        ],
    )
    def k(pu_hbm, pup_hbm, pv_hbm, pvp_hbm, ng_hbm, ngp_hbm, tu_hbm, tv_hbm,
          out_pos, out_neg,
          pu_idx, pv_idx, ng_idx, par_u, par_v, par_ng,
          u_rows, buf_a, buf_b, s_pos, s_neg, sem_a, sem_b):
        wid = lax.axis_index("s") * NC + lax.axis_index("c")
        iota = lax.iota(jnp.int32, L)
        lrow_u = (wid & 1) * 4

        pltpu.sync_copy(pu_hbm.at[pl.ds(pl.multiple_of((wid >> 1) * 8, 8),
                                        8)], pu_idx)
        pltpu.sync_copy(pv_hbm.at[pl.ds(pl.multiple_of((wid >> 1) * 8, 8),
                                        8)], pv_idx)
        pltpu.sync_copy(ng_hbm.at[pl.ds(pl.multiple_of(wid * 80, 8), 80)],
                        ng_idx)
        pltpu.sync_copy(pup_hbm, par_u)
        pltpu.sync_copy(pvp_hbm, par_v)
        pltpu.sync_copy(ngp_hbm.at[pl.ds(pl.multiple_of((wid >> 3) * 40, 8),
                                         40)], par_ng)

        def par16(buf, row, col):
            pw = plsc.load_gather(buf, [jnp.full((L,), row, jnp.int32),
                                        jnp.full((L,), col, jnp.int32)])
            return ((pw >> iota) & 1) << 6

        def buf_sem(buf):
            return sem_a if buf is buf_a else sem_b

        def round_issue(t_hbm, idx_ref, row, buf):
            return pltpu.async_copy(t_hbm.at[idx_ref.at[row]], buf,
                                    buf_sem(buf))

        def round_wait(t_hbm, idx_ref, row, buf):
            pltpu.make_async_copy(t_hbm.at[idx_ref.at[row]], buf,
                                  buf_sem(buf)).wait()

        def u_round_compact(r, buf):
            def grp(g, carry):
                s_loc = g * L + iota
                gu = wid * 32 + r * 8 + g
                p16 = par16(par_u, gu >> 7, gu & 127)

                def dstep(dd):
                    rd = (dd + iota) & (D - 1)
                    val = plsc.load_gather(buf, [s_loc, p16 + rd])
                    plsc.store_scatter(u_rows, [r * 128 + s_loc, rd], val)

                plsc.parallel_loop(0, D, unroll=8)(dstep)
                return carry

            lax.fori_loop(0, 8, grp, 0)

        round_issue(tu_hbm, pu_idx, lrow_u + 0, buf_a)
        round_issue(tu_hbm, pu_idx, lrow_u + 1, buf_b)
        round_wait(tu_hbm, pu_idx, lrow_u + 0, buf_a)
        u_round_compact(0, buf_a)
        round_wait(tu_hbm, pu_idx, lrow_u + 1, buf_b)
        u_round_compact(1, buf_b)
        round_issue(tu_hbm, pu_idx, lrow_u + 2, buf_a)
        round_issue(tu_hbm, pu_idx, lrow_u + 3, buf_b)
        round_wait(tu_hbm, pu_idx, lrow_u + 2, buf_a)
        u_round_compact(2, buf_a)
        round_wait(tu_hbm, pu_idx, lrow_u + 3, buf_b)
        u_round_compact(3, buf_b)

        def dot_pass(buf, urow_of, p16_of, sref, srow):
            def blk(j, carry):
                s_loc = j * L + iota
                urows = urow_of(j, s_loc)
                p16 = p16_of(j)

                def dstep(d4, accs):
                    out = []
                    for u in range(4):
                        rd = (d4 * 4 + u + iota) & (D - 1)
                        uc = plsc.load_gather(u_rows, [urows, rd])
                        oc = plsc.load_gather(buf, [s_loc, p16 + rd])
                        out.append(accs[u] + uc * oc)
                    return tuple(out)

                accs = lax.fori_loop(
                    0, D // 4, dstep,
                    tuple(jnp.zeros((L,), jnp.float32) for _ in range(4)))
                acc = (accs[0] + accs[1]) + (accs[2] + accs[3])
                plsc.store_scatter(sref,
                                   [jnp.full((L,), 0, jnp.int32) + srow,
                                    j * L + iota], acc)
                return carry

            lax.fori_loop(0, 8, blk, 0)

        def pos_quarter(buf, q):
            def p16_of(j):
                gv = wid * 32 + q * 8 + j
                return par16(par_v, gv >> 7, gv & 127)

            dot_pass(buf, lambda j, s_loc: q * 128 + s_loc, p16_of,
                     s_pos, q)

        round_issue(tv_hbm, pv_idx, lrow_u + 0, buf_a)
        round_issue(tv_hbm, pv_idx, lrow_u + 1, buf_b)
        round_wait(tv_hbm, pv_idx, lrow_u + 0, buf_a)
        pos_quarter(buf_a, 0)
        round_issue(tv_hbm, pv_idx, lrow_u + 2, buf_a)
        round_wait(tv_hbm, pv_idx, lrow_u + 1, buf_b)
        pos_quarter(buf_b, 1)
        round_issue(tv_hbm, pv_idx, lrow_u + 3, buf_b)
        round_wait(tv_hbm, pv_idx, lrow_u + 2, buf_a)
        pos_quarter(buf_a, 2)
        round_issue(tv_hbm, ng_idx, 0, buf_a)
        round_wait(tv_hbm, pv_idx, lrow_u + 3, buf_b)
        pos_quarter(buf_b, 3)
        pad = jnp.full((L,), 40.0, jnp.float32)
        for r in range(4, 8):
            for c in range(8):
                plsc.store_scatter(s_pos,
                                   [jnp.full((L,), r, jnp.int32),
                                    c * L + iota], pad)
        pltpu.sync_copy(s_pos,
                        out_pos.at[pl.ds(pl.multiple_of(wid * 8, 8), 8)])

        def neg_compute(buf, hc):
            def p16_of(j):
                g_w = hc * 8 + j
                return par16(par_ng, (wid & 7) * 5 + (g_w >> 7), g_w & 127)

            dot_pass(buf, lambda j, s_loc: (hc * 128 + s_loc) // K,
                     p16_of, s_neg, hc % 40)

        def pair(i, carry):
            hc0 = 2 * i
            hc1 = hc0 + 1
            round_issue(tv_hbm, ng_idx, hc1, buf_b)
            round_wait(tv_hbm, ng_idx, hc0, buf_a)
            neg_compute(buf_a, hc0)
            hc2 = jnp.minimum(hc0 + 2, NHC - 2)
            round_issue(tv_hbm, ng_idx, hc2, buf_a)
            round_wait(tv_hbm, ng_idx, hc1, buf_b)
            neg_compute(buf_b, hc1)

            @pl.when(hc1 == NHC // 2 - 1)
            def _():
                pltpu.sync_copy(
                    s_neg,
                    out_neg.at[pl.ds(pl.multiple_of(wid * 80, 8), 40)])

            return carry

        lax.fori_loop(0, NHC // 2, pair, 0)
        round_wait(tv_hbm, ng_idx, NHC - 2, buf_a)
        pltpu.sync_copy(
            s_neg,
            out_neg.at[pl.ds(pl.multiple_of(wid * 80 + 40, 8), 40)])

    return k(pu2, par_u, pv2, par_v, ng2, par_ng, tu, tv)


@functools.partial(jax.jit, static_argnames=("B",))
def _tc_loss(pos_score, neg_score, *, B):
    def body(p_ref, n_ref, o_ref):
        def log_sig(x):
            return jnp.minimum(x, 0.0) - jnp.log1p(jnp.exp(-jnp.abs(x)))

        tot = jnp.sum(log_sig(p_ref[...])) + jnp.sum(log_sig(-n_ref[...]))
        o_ref[0, 0] = -tot / B

    out = pl.pallas_call(
        body,
        out_shape=jax.ShapeDtypeStruct((1, 1), jnp.float32),
        out_specs=pl.BlockSpec(memory_space=pltpu.SMEM),
    )(pos_score, neg_score)
    return out[0, 0]


def _pack16(bits):
    w = bits.reshape(-1, 16) << jnp.arange(16, dtype=jnp.int32)
    return jnp.sum(w, axis=1).astype(jnp.int32)


def kernel(pos_u, pos_v, neg_v, embed_u, embed_v):
    V, D = embed_u.shape
    B, K = neg_v.shape
    ntail = (V % 256) // 2
    tu, tv = _sc_transpose(
        embed_u.T, embed_v.T,
        embed_u[V - 2 * ntail:, :].reshape(ntail, 2 * D),
        embed_v[V - 2 * ntail:, :].reshape(ntail, 2 * D), V=V, D=D)
    pu = pos_u.astype(jnp.int32)
    pv = pos_v.astype(jnp.int32)
    nf = neg_v.astype(jnp.int32).reshape(-1)
    pos_s, neg_s = _sc_scores(
        (pu >> 1).reshape(B // 128, 128), _pack16(pu & 1).reshape(-1, 128),
        (pv >> 1).reshape(B // 128, 128), _pack16(pv & 1).reshape(-1, 128),
        (nf >> 1).reshape(B * K // 128, 128),
        _pack16(nf & 1).reshape(-1, 128),
        tu, tv, B=B, K=K, D=D)
    return _tc_loss(pos_s, neg_s, B=B)


# speedup vs baseline: 2.7430x; 1.2761x over previous
import functools

import jax
import jax.numpy as jnp
from jax import lax
from jax.experimental import pallas as pl
from jax.experimental.pallas import tpu as pltpu
from jax.experimental.pallas import tpu_sc as plsc

NC = 2
NS = 16
NW = NC * NS
L = 16

_SC_PARAMS = pltpu.CompilerParams(needs_layout_passes=False,
                                  use_tc_tiling_on_sc=True)


@functools.partial(jax.jit, static_argnames=("V", "D"))
def _sc_transpose(eu_t, ev_t, tail_u, tail_v, *, V, D):
    WB = 256
    NB = V // WB
    NBPW = (NB + NW - 1) // NW
    mesh = plsc.VectorSubcoreMesh(core_axis_name="c", subcore_axis_name="s")

    @functools.partial(
        pl.kernel,
        out_type=(jax.ShapeDtypeStruct((V // 2, 128), jnp.float32),
                  jax.ShapeDtypeStruct((V // 2, 128), jnp.float32)),
        mesh=mesh,
        compiler_params=_SC_PARAMS,
        scratch_types=[
            pltpu.VMEM((D, WB), jnp.float32),
            pltpu.VMEM((D, WB), jnp.float32),
            pltpu.VMEM((WB // 2, 128), jnp.float32),
            pltpu.VMEM((WB // 2, 128), jnp.float32),
            pltpu.SemaphoreType.DMA,
            pltpu.SemaphoreType.DMA,
        ],
    )
    def k(eu_hbm, ev_hbm, tlu_hbm, tlv_hbm, tu_hbm, tv_hbm,
          in0, in1, o0, o1, sin, sout):
        wid = lax.axis_index("s") * NC + lax.axis_index("c")
        iota = lax.iota(jnp.int32, L)

        def transpose_block(src, dst, nw):
            def grp(g, carry):
                wl = g * L + iota
                prow = wl >> 1
                pcol = (wl & 1) << 6

                def dstep(dd):
                    rd = (dd + iota) & (D - 1)
                    val = plsc.load_gather(src, [rd, wl])
                    plsc.store_scatter(dst, [prow, pcol + rd], val)

                plsc.parallel_loop(0, D, unroll=8)(dstep)
                return carry

            lax.fori_loop(0, nw // L, grp, 0)

        for t_hbm, o_hbm in ((eu_hbm, tu_hbm), (ev_hbm, tv_hbm)):
            start = wid * NBPW
            end = jnp.minimum(start + NBPW, NB)
            npair = (end - start + 1) // 2

            def in_copy(bb, buf):
                off = pl.multiple_of(bb * WB, WB)
                return pltpu.make_async_copy(
                    t_hbm.at[:, pl.ds(off, WB)], buf, sin)

            def out_copy(bb, buf):
                off = pl.multiple_of(bb * (WB // 2), WB // 2)
                return pltpu.make_async_copy(
                    buf, o_hbm.at[pl.ds(off, WB // 2)], sout)

            in_copy(start, in0).start()
            in_copy(jnp.minimum(start + 1, end - 1), in1).start()

            def pair(i, carry):
                b0 = start + 2 * i
                b1 = jnp.minimum(b0 + 1, end - 1)
                nb0 = jnp.minimum(b0 + 2, end - 1)
                nb1 = jnp.minimum(b0 + 3, end - 1)

                @pl.when(i > 0)
                def _():
                    out_copy(b0 - 2, o0).wait()
                    out_copy(b1 - 2, o1).wait()

                in_copy(b0, in0).wait()
                transpose_block(in0, o0, WB)
                out_copy(b0, o0).start()
                in_copy(nb0, in0).start()
                in_copy(b1, in1).wait()
                transpose_block(in1, o1, WB)
                out_copy(b1, o1).start()
                in_copy(nb1, in1).start()
                return carry

            lax.fori_loop(0, npair, pair, 0)
            last0 = start + 2 * (npair - 1)
            out_copy(last0, o0).wait()
            out_copy(jnp.minimum(last0 + 1, end - 1), o1).wait()
            in_copy(jnp.minimum(last0 + 2, end - 1), in0).wait()
            in_copy(jnp.minimum(last0 + 3, end - 1), in1).wait()

        @pl.when(wid == NW - 1)
        def _():
            pltpu.sync_copy(tlv_hbm, o0.at[pl.ds(0, 32)])
            pltpu.sync_copy(o0.at[pl.ds(0, 32)],
                            tv_hbm.at[pl.ds(V // 2 - 32, 32)])

        @pl.when(wid == NW - 2)
        def _():
            pltpu.sync_copy(tlu_hbm, o0.at[pl.ds(0, 32)])
            pltpu.sync_copy(o0.at[pl.ds(0, 32)],
                            tu_hbm.at[pl.ds(V // 2 - 32, 32)])

    return k(eu_t, ev_t, tail_u, tail_v)


@functools.partial(jax.jit, static_argnames=("B", "K", "D"))
def _sc_scores(pu2, par_u, pv2, par_v, ng2, par_ng, tu, tv, *, B, K, D):
    RPW = B // NW
    NHC = RPW * K // 128
    mesh = plsc.VectorSubcoreMesh(core_axis_name="c", subcore_axis_name="s")

    @functools.partial(
        pl.kernel,
        out_type=(jax.ShapeDtypeStruct((NW * 8, 128), jnp.float32),
                  jax.ShapeDtypeStruct((NW * 80, 128), jnp.float32)),
        mesh=mesh,
        compiler_params=_SC_PARAMS,
        scratch_types=[
            pltpu.VMEM((8, 128), jnp.int32),
            pltpu.VMEM((8, 128), jnp.int32),
            pltpu.VMEM((80, 128), jnp.int32),
            pltpu.VMEM((8, 128), jnp.int32),
            pltpu.VMEM((8, 128), jnp.int32),
            pltpu.VMEM((40, 128), jnp.int32),
            pltpu.VMEM((RPW, D), jnp.float32),
            pltpu.VMEM((128, 128), jnp.float32),
            pltpu.VMEM((128, 128), jnp.float32),
            pltpu.VMEM((8, 128), jnp.float32),
            pltpu.VMEM((40, 128), jnp.float32),
            pltpu.SemaphoreType.DMA,
            pltpu.SemaphoreType.DMA,
        ],
    )
    def k(pu_hbm, pup_hbm, pv_hbm, pvp_hbm, ng_hbm, ngp_hbm, tu_hbm, tv_hbm,
          out_pos, out_neg,
          pu_idx, pv_idx, ng_idx, par_u, par_v, par_ng,
          u_rows, buf_a, buf_b, s_pos, s_neg, sem_a, sem_b):
        wid = lax.axis_index("s") * NC + lax.axis_index("c")
        iota = lax.iota(jnp.int32, L)
        lrow_u = (wid & 1) * 4

        pltpu.sync_copy(pu_hbm.at[pl.ds(pl.multiple_of((wid >> 1) * 8, 8),
                                        8)], pu_idx)
        pltpu.sync_copy(pv_hbm.at[pl.ds(pl.multiple_of((wid >> 1) * 8, 8),
                                        8)], pv_idx)
        pltpu.sync_copy(ng_hbm.at[pl.ds(pl.multiple_of(wid * 80, 8), 80)],
                        ng_idx)
        pltpu.sync_copy(pup_hbm, par_u)
        pltpu.sync_copy(pvp_hbm, par_v)
        pltpu.sync_copy(ngp_hbm.at[pl.ds(pl.multiple_of((wid >> 3) * 40, 8),
                                         40)], par_ng)

        def par16(buf, row, col):
            pw = plsc.load_gather(buf, [jnp.full((L,), row, jnp.int32),
                                        jnp.full((L,), col, jnp.int32)])
            return ((pw >> iota) & 1) << 6

        def buf_sem(buf):
            return sem_a if buf is buf_a else sem_b

        def round_issue(t_hbm, idx_ref, row, buf):
            return pltpu.async_copy(t_hbm.at[idx_ref.at[row]], buf,
                                    buf_sem(buf))

        def round_wait(t_hbm, idx_ref, row, buf):
            pltpu.make_async_copy(t_hbm.at[idx_ref.at[row]], buf,
                                  buf_sem(buf)).wait()

        def u_round_compact(r, buf):
            def grp(g, carry):
                s_loc = g * L + iota
                gu = wid * 32 + r * 8 + g
                p16 = par16(par_u, gu >> 7, gu & 127)

                def dstep(dd):
                    rd = (dd + iota) & (D - 1)
                    val = plsc.load_gather(buf, [s_loc, p16 + rd])
                    plsc.store_scatter(u_rows, [r * 128 + s_loc, rd], val)

                plsc.parallel_loop(0, D, unroll=8)(dstep)
                return carry

            lax.fori_loop(0, 8, grp, 0)

        round_issue(tu_hbm, pu_idx, lrow_u + 0, buf_a)
        round_issue(tu_hbm, pu_idx, lrow_u + 1, buf_b)
        round_wait(tu_hbm, pu_idx, lrow_u + 0, buf_a)
        u_round_compact(0, buf_a)
        round_wait(tu_hbm, pu_idx, lrow_u + 1, buf_b)
        u_round_compact(1, buf_b)
        round_issue(tu_hbm, pu_idx, lrow_u + 2, buf_a)
        round_issue(tu_hbm, pu_idx, lrow_u + 3, buf_b)
        round_wait(tu_hbm, pu_idx, lrow_u + 2, buf_a)
        u_round_compact(2, buf_a)
        round_wait(tu_hbm, pu_idx, lrow_u + 3, buf_b)
        u_round_compact(3, buf_b)

        def dot_pass(buf, urow_of, p16_of, sref, srow):
            def blk(j, carry):
                s_loc = j * L + iota
                urows = urow_of(j, s_loc)
                p16 = p16_of(j)

                def dstep(d4, accs):
                    out = []
                    for u in range(4):
                        rd = (d4 * 4 + u + iota) & (D - 1)
                        uc = plsc.load_gather(u_rows, [urows, rd])
                        oc = plsc.load_gather(buf, [s_loc, p16 + rd])
                        out.append(accs[u] + uc * oc)
                    return tuple(out)

                accs = lax.fori_loop(
                    0, D // 4, dstep,
                    tuple(jnp.zeros((L,), jnp.float32) for _ in range(4)))
                acc = (accs[0] + accs[1]) + (accs[2] + accs[3])
                plsc.store_scatter(sref,
                                   [jnp.full((L,), 0, jnp.int32) + srow,
                                    j * L + iota], acc)
                return carry

            lax.fori_loop(0, 8, blk, 0)

        def pos_quarter(buf, q):
            def p16_of(j):
                gv = wid * 32 + q * 8 + j
                return par16(par_v, gv >> 7, gv & 127)

            dot_pass(buf, lambda j, s_loc: q * 128 + s_loc, p16_of,
                     s_pos, q)

        round_issue(tv_hbm, pv_idx, lrow_u + 0, buf_a)
        round_issue(tv_hbm, pv_idx, lrow_u + 1, buf_b)
        round_wait(tv_hbm, pv_idx, lrow_u + 0, buf_a)
        pos_quarter(buf_a, 0)
        round_issue(tv_hbm, pv_idx, lrow_u + 2, buf_a)
        round_wait(tv_hbm, pv_idx, lrow_u + 1, buf_b)
        pos_quarter(buf_b, 1)
        round_issue(tv_hbm, pv_idx, lrow_u + 3, buf_b)
        round_wait(tv_hbm, pv_idx, lrow_u + 2, buf_a)
        pos_quarter(buf_a, 2)
        round_issue(tv_hbm, ng_idx, 0, buf_a)
        round_wait(tv_hbm, pv_idx, lrow_u + 3, buf_b)
        pos_quarter(buf_b, 3)
        pad = jnp.full((L,), 40.0, jnp.float32)
        for r in range(4, 8):
            for c in range(8):
                plsc.store_scatter(s_pos,
                                   [jnp.full((L,), r, jnp.int32),
                                    c * L + iota], pad)
        pltpu.sync_copy(s_pos,
                        out_pos.at[pl.ds(pl.multiple_of(wid * 8, 8), 8)])

        def neg_compute(buf, hc):
            def p16_of(j):
                g_w = hc * 8 + j
                return par16(par_ng, (wid & 7) * 5 + (g_w >> 7), g_w & 127)

            dot_pass(buf, lambda j, s_loc: (hc * 128 + s_loc) // K,
                     p16_of, s_neg, hc % 40)

        def pair(i, carry):
            hc0 = 2 * i
            hc1 = hc0 + 1
            round_issue(tv_hbm, ng_idx, hc1, buf_b)
            round_wait(tv_hbm, ng_idx, hc0, buf_a)
            neg_compute(buf_a, hc0)
            hc2 = jnp.minimum(hc0 + 2, NHC - 2)
            round_issue(tv_hbm, ng_idx, hc2, buf_a)
            round_wait(tv_hbm, ng_idx, hc1, buf_b)
            neg_compute(buf_b, hc1)

            @pl.when(hc1 == NHC // 2 - 1)
            def _():
                pltpu.sync_copy(
                    s_neg,
                    out_neg.at[pl.ds(pl.multiple_of(wid * 80, 8), 40)])

            return carry

        lax.fori_loop(0, NHC // 2, pair, 0)
        round_wait(tv_hbm, ng_idx, NHC - 2, buf_a)
        pltpu.sync_copy(
            s_neg,
            out_neg.at[pl.ds(pl.multiple_of(wid * 80 + 40, 8), 40)])

    return k(pu2, par_u, pv2, par_v, ng2, par_ng, tu, tv)


@functools.partial(jax.jit, static_argnames=("B",))
def _tc_loss(pos_score, neg_score, *, B):
    def body(p_ref, n_ref, o_ref):
        def log_sig(x):
            return jnp.minimum(x, 0.0) - jnp.log1p(jnp.exp(-jnp.abs(x)))

        tot = jnp.sum(log_sig(p_ref[...])) + jnp.sum(log_sig(-n_ref[...]))
        o_ref[0, 0] = -tot / B

    out = pl.pallas_call(
        body,
        out_shape=jax.ShapeDtypeStruct((1, 1), jnp.float32),
        out_specs=pl.BlockSpec(memory_space=pltpu.SMEM),
    )(pos_score, neg_score)
    return out[0, 0]


def _pack16(bits):
    w = bits.reshape(-1, 16) << jnp.arange(16, dtype=jnp.int32)
    return jnp.sum(w, axis=1).astype(jnp.int32)


def kernel(pos_u, pos_v, neg_v, embed_u, embed_v):
    V, D = embed_u.shape
    B, K = neg_v.shape
    ntail = (V % 256) // 2
    tu, tv = _sc_transpose(
        embed_u.T, embed_v.T,
        embed_u[V - 2 * ntail:, :].reshape(ntail, 2 * D),
        embed_v[V - 2 * ntail:, :].reshape(ntail, 2 * D), V=V, D=D)
    pu = pos_u.astype(jnp.int32)
    pv = pos_v.astype(jnp.int32)
    nf = neg_v.astype(jnp.int32).reshape(-1)
    pos_s, neg_s = _sc_scores(
        (pu >> 1).reshape(B // 128, 128), _pack16(pu & 1).reshape(-1, 128),
        (pv >> 1).reshape(B // 128, 128), _pack16(pv & 1).reshape(-1, 128),
        (nf >> 1).reshape(B * K // 128, 128),
        _pack16(nf & 1).reshape(-1, 128),
        tu, tv, B=B, K=K, D=D)
    return _tc_loss(pos_s, neg_s, B=B)
